# baseline (speedup 1.0000x reference)
import functools

import jax
import jax.numpy as jnp
from jax import lax
from jax.experimental import pallas as pl
from jax.experimental.pallas import tpu as pltpu
from jax.experimental.pallas import tpu_sc as plsc

DIM = 64
B = 16384
LAMBD = 0.1

_info = plsc.get_sparse_core_info()
_NC, _NS, _L = _info.num_cores, _info.num_subcores, _info.num_lanes
_NW = _NC * _NS
_PER_W = B // _NW
_CHUNK = 256
_NCHUNK = _PER_W // _CHUNK


def _sc_dots(sources, contexts, personas, pure_sources,
             node_emb, noise_emb, base_emb):
    mesh = plsc.VectorSubcoreMesh(core_axis_name="c", subcore_axis_name="s")

    @functools.partial(
        pl.kernel,
        mesh=mesh,
        compiler_params=pltpu.CompilerParams(
            needs_layout_passes=False, use_tc_tiling_on_sc=False),
        out_type=[jax.ShapeDtypeStruct((B,), jnp.float32) for _ in range(4)],
        scratch_types=[
            pltpu.VMEM((_PER_W,), jnp.int32),
            pltpu.VMEM((_PER_W,), jnp.int32),
            pltpu.VMEM((_PER_W,), jnp.int32),
            pltpu.VMEM((_PER_W,), jnp.int32),
            pltpu.VMEM((_CHUNK, DIM), jnp.float32),
            pltpu.VMEM((_CHUNK, DIM), jnp.float32),
            pltpu.VMEM((_CHUNK, DIM), jnp.float32),
            pltpu.VMEM((_CHUNK, DIM), jnp.float32),
            pltpu.VMEM((_PER_W,), jnp.float32),
            pltpu.VMEM((_PER_W,), jnp.float32),
            pltpu.VMEM((_PER_W,), jnp.float32),
            pltpu.VMEM((_PER_W,), jnp.float32),
            pltpu.SemaphoreType.DMA,
        ],
    )
    def body(src_hbm, ctx_hbm, psrc_hbm, pers_hbm,
             node_hbm, noise_hbm, base_hbm,
             dab_hbm, daa_hbm, dbb_hbm, dr_hbm,
             src_v, ctx_v, psrc_v, pers_v,
             ra, rb, rc, rd,
             dab_v, daa_v, dbb_v, dr_v, sem):
        wid = lax.axis_index("s") * _NC + lax.axis_index("c")
        base = wid * _PER_W
        lane = lax.iota(jnp.int32, 16)
        pltpu.sync_copy(src_hbm.at[pl.ds(base, _PER_W)], src_v)
        pltpu.sync_copy(ctx_hbm.at[pl.ds(base, _PER_W)], ctx_v)
        pltpu.sync_copy(psrc_hbm.at[pl.ds(base, _PER_W)], psrc_v)
        pltpu.sync_copy(pers_hbm.at[pl.ds(base, _PER_W)], pers_v)

        for c in range(_NCHUNK):
            off = c * _CHUNK
            pltpu.async_copy(node_hbm.at[src_v.at[pl.ds(off, _CHUNK)]], ra, sem).wait()
            pltpu.async_copy(noise_hbm.at[ctx_v.at[pl.ds(off, _CHUNK)]], rb, sem).wait()
            pltpu.async_copy(node_hbm.at[psrc_v.at[pl.ds(off, _CHUNK)]], rc, sem).wait()
            pltpu.async_copy(base_hbm.at[pers_v.at[pl.ds(off, _CHUNK)]], rd, sem).wait()

            def grp_body(g, carry):
                dab_t = jnp.zeros((16,), jnp.float32)
                daa_t = jnp.zeros((16,), jnp.float32)
                dbb_t = jnp.zeros((16,), jnp.float32)
                dr_t = jnp.zeros((16,), jnp.float32)
                for r in range(16):
                    row = g * 16 + r
                    a0 = ra[row, pl.ds(0, 16)]
                    a1 = ra[row, pl.ds(16, 16)]
                    a2 = ra[row, pl.ds(32, 16)]
                    a3 = ra[row, pl.ds(48, 16)]
                    b0 = rb[row, pl.ds(0, 16)]
                    b1 = rb[row, pl.ds(16, 16)]
                    b2 = rb[row, pl.ds(32, 16)]
                    b3 = rb[row, pl.ds(48, 16)]
                    pab = a0 * b0 + a1 * b1 + a2 * b2 + a3 * b3
                    paa = a0 * a0 + a1 * a1 + a2 * a2 + a3 * a3
                    pbb = b0 * b0 + b1 * b1 + b2 * b2 + b3 * b3
                    c0 = rc[row, pl.ds(0, 16)]
                    c1 = rc[row, pl.ds(16, 16)]
                    c2 = rc[row, pl.ds(32, 16)]
                    c3 = rc[row, pl.ds(48, 16)]
                    d0 = rd[row, pl.ds(0, 16)]
                    d1 = rd[row, pl.ds(16, 16)]
                    d2 = rd[row, pl.ds(32, 16)]
                    d3 = rd[row, pl.ds(48, 16)]
                    pr = c0 * d0 + c1 * d1 + c2 * d2 + c3 * d3
                    m = lane == r
                    dab_t = jnp.where(m, jnp.sum(pab), dab_t)
                    daa_t = jnp.where(m, jnp.sum(paa), daa_t)
                    dbb_t = jnp.where(m, jnp.sum(pbb), dbb_t)
                    dr_t = jnp.where(m, jnp.sum(pr), dr_t)
                goff = off + g * 16
                dab_v[pl.ds(goff, 16)] = dab_t
                daa_v[pl.ds(goff, 16)] = daa_t
                dbb_v[pl.ds(goff, 16)] = dbb_t
                dr_v[pl.ds(goff, 16)] = dr_t
                return carry

            lax.fori_loop(0, _CHUNK // 16, grp_body, 0)

        pltpu.sync_copy(dab_v, dab_hbm.at[pl.ds(base, _PER_W)])
        pltpu.sync_copy(daa_v, daa_hbm.at[pl.ds(base, _PER_W)])
        pltpu.sync_copy(dbb_v, dbb_hbm.at[pl.ds(base, _PER_W)])
        pltpu.sync_copy(dr_v, dr_hbm.at[pl.ds(base, _PER_W)])

    return body(sources, contexts, pure_sources, personas,
                node_emb, noise_emb, base_emb)


def _finish_body(dab_ref, daa_ref, dbb_ref, dr_ref, t_ref, out_ref):
    dab = dab_ref[...]
    daa = daa_ref[...]
    dbb = dbb_ref[...]
    dr = dr_ref[...]
    t = t_ref[...]
    na = jnp.maximum(jnp.sqrt(daa), 1e-12)
    nb = jnp.maximum(jnp.sqrt(dbb), 1e-12)
    x = dab / (na * nb)
    s = jax.nn.sigmoid(x)
    main = t * jnp.log(s) + (1.0 - t) * jnp.log(1.0 - s)
    r = jax.nn.sigmoid(jnp.clip(dr, -15.0, 15.0))
    out_ref[0, 0] = -jnp.mean(main) - LAMBD * jnp.mean(jnp.log(r))


def kernel(sources, contexts, targets, personas, pure_sources,
           node_embedding, node_noise_embedding, base_node_embedding):
    src = sources.astype(jnp.int32)
    ctx = contexts.astype(jnp.int32)
    pers = personas.astype(jnp.int32)
    psrc = pure_sources.astype(jnp.int32)
    dab, daa, dbb, dr = _sc_dots(src, ctx, pers, psrc,
                                 node_embedding, node_noise_embedding,
                                 base_node_embedding)
    loss = pl.pallas_call(
        _finish_body,
        out_shape=jax.ShapeDtypeStruct((1, 1), jnp.float32),
        out_specs=pl.BlockSpec(memory_space=pltpu.SMEM),
    )(dab.reshape(128, 128), daa.reshape(128, 128),
      dbb.reshape(128, 128), dr.reshape(128, 128),
      targets.reshape(128, 128))
    return loss.reshape(())
